# baseline (speedup 1.0000x reference)
import jax
import jax.numpy as jnp
from jax import lax
from jax.experimental import pallas as pl
from jax.experimental.pallas import tpu as pltpu
from jax.experimental.pallas import tpu_sc as plsc

USER_NUM = 25000
ITEM_NUM = 25000
N_NODES = USER_NUM + ITEM_NUM
N_EDGES = 800000
EMB = 64
QC = 16
NQ = 4
N_LAYERS = 3

NC = 2
NS = 16
CH = 128
JJ = 8
GROUP = JJ * CH
G = 49
EPT = G * GROUP
EP = NS * EPT
PT = N_NODES // NS


def _spmm_body(table, colh, lidxh, valh, zrows, out,
               colv, lidxv, valv, gbuf, acc, gsem, ssem):
    c = lax.axis_index("c")
    s = lax.axis_index("s")

    row_base = s * (EPT // CH)
    flat_base = s * EPT

    def drain_scatter(j):
        pltpu.make_async_copy(
            gbuf.at[pl.ds(j * CH, CH)],
            acc.at[pl.ds(j * CH, CH)], ssem.at[j]).wait()

    for p in range(2):
        q = c * 2 + p

        pltpu.sync_copy(zrows, acc.at[pl.ds(s * PT, PT)])
        plsc.subcore_barrier()

        def group_loop(g, _):
            @pl.when(g > 0)
            def _():
                for j in range(JJ):
                    drain_scatter(j)

            roff = row_base + g * JJ
            foff = flat_base + g * GROUP
            pltpu.sync_copy(colh.at[q, pl.ds(roff, JJ)], colv)
            pltpu.sync_copy(lidxh.at[pl.ds(roff, JJ)], lidxv)
            pltpu.sync_copy(valh.at[pl.ds(foff, GROUP)], valv)

            gds = [pltpu.async_copy(table.at[colv.at[j]],
                                    gbuf.at[pl.ds(j * CH, CH)], gsem.at[j])
                   for j in range(JJ)]

            for j in range(JJ):
                gds[j].wait()

                def e_loop(e16, _):
                    vv = valv[pl.ds(j * CH + e16 * 16, 16)]
                    for l in range(16):
                        r = j * CH + e16 * 16 + l
                        gbuf[r, :] = gbuf[r, :] * vv[l]
                    return 0

                lax.fori_loop(0, CH // 16, e_loop, 0)
                pltpu.async_copy(gbuf.at[pl.ds(j * CH, CH)],
                                 acc.at[lidxv.at[j]], ssem.at[j], add=True)
            return 0

        lax.fori_loop(0, G, group_loop, 0)
        for j in range(JJ):
            drain_scatter(j)
        plsc.subcore_barrier()

        pltpu.sync_copy(acc.at[pl.ds(s * PT, PT)],
                        out.at[pl.ds(q * N_NODES + s * PT, PT)])


_spmm = pl.kernel(
    _spmm_body,
    out_type=jax.ShapeDtypeStruct((NQ * N_NODES, QC), jnp.float32),
    mesh=plsc.VectorSubcoreMesh(core_axis_name="c", subcore_axis_name="s"),
    scratch_types=[
        pltpu.VMEM((JJ, CH), jnp.int32),
        pltpu.VMEM((JJ, CH), jnp.int32),
        pltpu.VMEM((GROUP,), jnp.float32),
        pltpu.VMEM((GROUP, QC), jnp.float32),
        pltpu.VMEM_SHARED((N_NODES, QC), jnp.float32),
        pltpu.SemaphoreType.DMA((JJ,)),
        pltpu.SemaphoreType.DMA((JJ,)),
    ],
    compiler_params=pltpu.CompilerParams(use_tc_tiling_on_sc=False),
)


def _mean3_body(a, b, c, o):
    o[...] = (a[...] + b[...] + c[...]) * (1.0 / 3.0)


def _mean3(a, b, c):
    rows = NQ * N_NODES * QC // 128
    a = a.reshape(rows, 128)
    b = b.reshape(rows, 128)
    c = c.reshape(rows, 128)
    blk = 1000
    spec = pl.BlockSpec((blk, 128), lambda i: (i, 0))
    out = pl.pallas_call(
        _mean3_body,
        grid=(rows // blk,),
        in_specs=[spec, spec, spec],
        out_specs=spec,
        out_shape=jax.ShapeDtypeStruct((rows, 128), jnp.float32),
    )(a, b, c)
    return out.reshape(NQ * N_NODES, QC)


def kernel(user_emb, item_emb, adj_values, adj_indices):
    row = adj_indices[0].astype(jnp.int32)
    col = adj_indices[1].astype(jnp.int32)
    val = adj_values.astype(jnp.float32)
    row, col, val = jax.lax.sort((row, col, val), num_keys=1)

    pad = EP - N_EDGES
    colp = jnp.concatenate([col, jnp.zeros((pad,), jnp.int32)])
    lidxp = jnp.concatenate([row, jnp.zeros((pad,), jnp.int32)])
    valp = jnp.concatenate([val, jnp.zeros((pad,), jnp.float32)])
    offs = (jnp.arange(NQ, dtype=jnp.int32) * N_NODES)[:, None]
    colh = (colp[None, :] + offs).reshape(NQ, EP // CH, CH)
    lidxh = lidxp.reshape(EP // CH, CH)

    ego = jnp.concatenate([user_emb, item_emb], axis=0)
    table = jnp.concatenate(
        [ego[:, 0:16], ego[:, 16:32], ego[:, 32:48], ego[:, 48:64]], axis=0)

    zrows = jnp.zeros((PT, QC), jnp.float32)

    layers = []
    for _ in range(N_LAYERS):
        table = _spmm(table, colh, lidxh, valp, zrows)
        layers.append(table)

    m = _mean3(*layers)

    user = jnp.concatenate(
        [m[i * N_NODES:i * N_NODES + USER_NUM] for i in range(NQ)], axis=1)
    item = jnp.concatenate(
        [m[i * N_NODES + USER_NUM:(i + 1) * N_NODES] for i in range(NQ)],
        axis=1)
    return (user, item)


# speedup vs baseline: 1.1444x; 1.1444x over previous
import jax
import jax.numpy as jnp
from jax import lax
from jax.experimental import pallas as pl
from jax.experimental.pallas import tpu as pltpu
from jax.experimental.pallas import tpu_sc as plsc

USER_NUM = 25000
ITEM_NUM = 25000
N_NODES = USER_NUM + ITEM_NUM
N_EDGES = 800000
EMB = 64
N_LAYERS = 3

NC = 2
NS = 16
NB = 4
BK = N_NODES // NB
CH = 128
JJ = 4
NCH = N_EDGES // CH
NCHP = NCH + JJ
UNIT = 625
NU = BK // UNIT
SUB = 125
NSUB = UNIT // SUB


def _body(table, rowh, colh, valh, offh, zrows, l1, l2, out,
          offv, colv, rowv, valv, gbuf, acc, gsem, ssem):
    c = lax.axis_index("c")
    s = lax.axis_index("s")

    pltpu.sync_copy(offh, offv)
    ov = offv[pl.ds(0, 16)]
    offs = [ov[i] for i in range(NB + 1)]
    is_final = ov[NB + 1]

    if True:

        def drain(j):
            pltpu.make_async_copy(
                gbuf.at[pl.ds(j * CH, CH)],
                acc.at[pl.ds(j * CH, CH)], ssem.at[j]).wait()

        for p in range(2):
            b = c * 2 + p
            lo = jnp.where(c == 0, offs[p], offs[2 + p])
            hi = jnp.where(c == 0, offs[p + 1], offs[3 + p])
            base_row = b * BK

            pltpu.sync_copy(zrows, acc.at[pl.ds(s * UNIT, UNIT)])

            @pl.when(s < NU - NS)
            def _():
                pltpu.sync_copy(zrows, acc.at[pl.ds((NS + s) * UNIT, UNIT)])

            plsc.subcore_barrier()

            c0 = lo // CH
            c1 = (hi + CH - 1) // CH
            nch = c1 - c0
            quota = (nch + NS - 1) // NS
            my_lo = c0 + s * quota
            my_n = jnp.clip(nch - s * quota, 0, quota)
            ngroups = (my_n + JJ - 1) // JJ
            lo_s = jnp.maximum(lo, my_lo * CH)
            hi_s = jnp.minimum(hi, (my_lo + my_n) * CH)

            def group_loop(g, _):
                @pl.when(g > 0)
                def _():
                    for j in range(JJ):
                        drain(j)

                bc = my_lo + g * JJ
                pltpu.sync_copy(colh.at[pl.ds(bc, JJ)], colv)
                pltpu.sync_copy(rowh.at[pl.ds(bc, JJ)], rowv)
                pltpu.sync_copy(valh.at[pl.ds(bc, JJ)], valv)

                gds = [pltpu.async_copy(table.at[colv.at[j]],
                                        gbuf.at[pl.ds(j * CH, CH)],
                                        gsem.at[j])
                       for j in range(JJ)]

                for j in range(JJ):
                    gds[j].wait()
                    ebase = (bc + j) * CH

                    def e_loop(e16, _):
                        eidx = ebase + e16 * 16 + lax.iota(jnp.int32, 16)
                        m = (eidx >= lo_s) & (eidx < hi_s)
                        vv = jnp.where(m, valv[j, pl.ds(e16 * 16, 16)], 0.0)
                        rr = jnp.where(
                            m, rowv[j, pl.ds(e16 * 16, 16)] - base_row, 0)
                        rowv[j, pl.ds(e16 * 16, 16)] = rr
                        for l in range(16):
                            r = j * CH + e16 * 16 + l
                            gbuf[r, :] = gbuf[r, :] * vv[l]
                        return 0

                    lax.fori_loop(0, CH // 16, e_loop, 0)
                    pltpu.async_copy(gbuf.at[pl.ds(j * CH, CH)],
                                     acc.at[rowv.at[j]], ssem.at[j],
                                     add=True)
                return 0

            lax.fori_loop(0, ngroups, group_loop, 0)

            @pl.when(ngroups > 0)
            def _():
                for j in range(JJ):
                    drain(j)

            plsc.subcore_barrier()

            def fused_unit(u):
                for k in range(NSUB):
                    o = u * UNIT + k * SUB
                    pltpu.sync_copy(acc.at[pl.ds(o, SUB)],
                                    gbuf.at[pl.ds(0, SUB)])
                    pltpu.sync_copy(l1.at[pl.ds(base_row + o, SUB)],
                                    gbuf.at[pl.ds(SUB, SUB)])
                    pltpu.sync_copy(l2.at[pl.ds(base_row + o, SUB)],
                                    gbuf.at[pl.ds(2 * SUB, SUB)])

                    def r_loop(r, _):
                        gbuf[r, :] = (gbuf[r, :] + gbuf[SUB + r, :] +
                                      gbuf[2 * SUB + r, :]) * (1.0 / 3.0)
                        return 0

                    lax.fori_loop(0, SUB, r_loop, 0)
                    pltpu.sync_copy(gbuf.at[pl.ds(0, SUB)],
                                    out.at[pl.ds(base_row + o, SUB)])

            def plain_unit(u):
                pltpu.sync_copy(
                    acc.at[pl.ds(u * UNIT, UNIT)],
                    out.at[pl.ds(base_row + u * UNIT, UNIT)])

            fin = is_final != 0
            nfin = jnp.logical_not(fin)

            @pl.when(fin)
            def _():
                fused_unit(s)

            @pl.when(fin & (s < NU - NS))
            def _():
                fused_unit(NS + s)

            @pl.when(nfin)
            def _():
                plain_unit(s)

            @pl.when(nfin & (s < NU - NS))
            def _():
                plain_unit(NS + s)


_spmm = pl.kernel(
    _body,
    out_type=jax.ShapeDtypeStruct((N_NODES, EMB), jnp.float32),
    mesh=plsc.VectorSubcoreMesh(core_axis_name="c", subcore_axis_name="s"),
    scratch_types=[
        pltpu.VMEM((16,), jnp.int32),
        pltpu.VMEM((JJ, CH), jnp.int32),
        pltpu.VMEM((JJ, CH), jnp.int32),
        pltpu.VMEM((JJ, CH), jnp.float32),
        pltpu.VMEM((JJ * CH, EMB), jnp.float32),
        pltpu.VMEM_SHARED((BK, EMB), jnp.float32),
        pltpu.SemaphoreType.DMA((JJ,)),
        pltpu.SemaphoreType.DMA((JJ,)),
    ],
    compiler_params=pltpu.CompilerParams(use_tc_tiling_on_sc=False),
)


def kernel(user_emb, item_emb, adj_values, adj_indices):
    row = adj_indices[0].astype(jnp.int32)
    col = adj_indices[1].astype(jnp.int32)
    val = adj_values.astype(jnp.float32)
    row, col, val = jax.lax.sort((row, col, val), num_keys=1)

    bounds = jnp.arange(0, N_NODES + BK, BK, dtype=jnp.int32)
    off = jnp.searchsorted(row, bounds).astype(jnp.int32)
    pad16 = jnp.zeros((10,), jnp.int32)
    offh = jnp.concatenate([off, jnp.zeros((1,), jnp.int32), pad16])
    offh_fin = jnp.concatenate([off, jnp.ones((1,), jnp.int32), pad16])

    pad = NCHP * CH - N_EDGES
    rowh = jnp.concatenate([row, jnp.zeros((pad,), jnp.int32)])
    colh = jnp.concatenate([col, jnp.zeros((pad,), jnp.int32)])
    valh = jnp.concatenate([val, jnp.zeros((pad,), jnp.float32)])
    rowh = rowh.reshape(NCHP, CH)
    colh = colh.reshape(NCHP, CH)
    valh = valh.reshape(NCHP, CH)

    table = jnp.concatenate([user_emb, item_emb], axis=0)
    zrows = jnp.zeros((UNIT, EMB), jnp.float32)

    t1 = _spmm(table, rowh, colh, valh, offh, zrows, table, table)
    t2 = _spmm(t1, rowh, colh, valh, offh, zrows, table, table)
    m = _spmm(t2, rowh, colh, valh, offh_fin, zrows, t1, t2)
    return (m[:USER_NUM], m[USER_NUM:])


# speedup vs baseline: 1.6827x; 1.4705x over previous
import jax
import jax.numpy as jnp
from jax import lax
from jax.experimental import pallas as pl
from jax.experimental.pallas import tpu as pltpu
from jax.experimental.pallas import tpu_sc as plsc

USER_NUM = 25000
ITEM_NUM = 25000
N_NODES = USER_NUM + ITEM_NUM
N_EDGES = 800000
EMB = 64
QC = 16
NQ = 4
N_LAYERS = 3

NC = 2
NS = 16
CH = 128
JJ = 8
GROUP = JJ * CH
G = 49
EPT = G * GROUP
EP = NS * EPT
PT = N_NODES // NS
SUB = 125
NSUB = PT // SUB


def _spmm_body(table, colh, lidxh, valh, flagh, zrows, l1, l2, out,
               flagv, colv, lidxv, valv, gbuf, acc, gsem, ssem):
    c = lax.axis_index("c")
    s = lax.axis_index("s")

    pltpu.sync_copy(flagh, flagv)
    is_final = flagv[pl.ds(0, 16)][0]

    row_base = s * (EPT // CH)
    flat_base = s * EPT

    def drain_scatter(j):
        pltpu.make_async_copy(
            gbuf.at[pl.ds(j * CH, CH)],
            acc.at[pl.ds(j * CH, CH)], ssem.at[j]).wait()

    for p in range(2):
        q = c * 2 + p

        pltpu.sync_copy(zrows, acc.at[pl.ds(s * PT, PT)])
        plsc.subcore_barrier()

        def group_loop(g, _):
            @pl.when(g > 0)
            def _():
                for j in range(JJ):
                    drain_scatter(j)

            roff = row_base + g * JJ
            foff = flat_base + g * GROUP
            pltpu.sync_copy(colh.at[q, pl.ds(roff, JJ)], colv)
            pltpu.sync_copy(lidxh.at[pl.ds(roff, JJ)], lidxv)
            pltpu.sync_copy(valh.at[pl.ds(foff, GROUP)], valv)

            gds = [pltpu.async_copy(table.at[colv.at[j]],
                                    gbuf.at[pl.ds(j * CH, CH)], gsem.at[j])
                   for j in range(JJ)]

            for j in range(JJ):
                gds[j].wait()

                def e_loop(e16, _):
                    vv = valv[pl.ds(j * CH + e16 * 16, 16)]
                    for l in range(16):
                        r = j * CH + e16 * 16 + l
                        gbuf[r, :] = gbuf[r, :] * vv[l]
                    return 0

                lax.fori_loop(0, CH // 16, e_loop, 0)
                pltpu.async_copy(gbuf.at[pl.ds(j * CH, CH)],
                                 acc.at[lidxv.at[j]], ssem.at[j], add=True)
            return 0

        lax.fori_loop(0, G, group_loop, 0)
        for j in range(JJ):
            drain_scatter(j)
        plsc.subcore_barrier()

        fin = is_final != 0

        @pl.when(fin)
        def _():
            def sub_loop(k, _):
                o = s * PT + k * SUB
                ho = q * N_NODES + s * PT + k * SUB
                pltpu.sync_copy(acc.at[pl.ds(o, SUB)],
                                gbuf.at[pl.ds(0, SUB)])
                pltpu.sync_copy(l1.at[pl.ds(ho, SUB)],
                                gbuf.at[pl.ds(SUB, SUB)])
                pltpu.sync_copy(l2.at[pl.ds(ho, SUB)],
                                gbuf.at[pl.ds(2 * SUB, SUB)])

                def r_loop(r, _):
                    gbuf[r, :] = (gbuf[r, :] + gbuf[SUB + r, :] +
                                  gbuf[2 * SUB + r, :]) * (1.0 / 3.0)
                    return 0

                lax.fori_loop(0, SUB, r_loop, 0)
                pltpu.sync_copy(gbuf.at[pl.ds(0, SUB)],
                                out.at[pl.ds(ho, SUB)])
                return 0

            lax.fori_loop(0, NSUB, sub_loop, 0)

        @pl.when(jnp.logical_not(fin))
        def _():
            pltpu.sync_copy(acc.at[pl.ds(s * PT, PT)],
                            out.at[pl.ds(q * N_NODES + s * PT, PT)])


_spmm = pl.kernel(
    _spmm_body,
    out_type=jax.ShapeDtypeStruct((NQ * N_NODES, QC), jnp.float32),
    mesh=plsc.VectorSubcoreMesh(core_axis_name="c", subcore_axis_name="s"),
    scratch_types=[
        pltpu.VMEM((16,), jnp.int32),
        pltpu.VMEM((JJ, CH), jnp.int32),
        pltpu.VMEM((JJ, CH), jnp.int32),
        pltpu.VMEM((GROUP,), jnp.float32),
        pltpu.VMEM((GROUP, QC), jnp.float32),
        pltpu.VMEM_SHARED((N_NODES, QC), jnp.float32),
        pltpu.SemaphoreType.DMA((JJ,)),
        pltpu.SemaphoreType.DMA((JJ,)),
    ],
    compiler_params=pltpu.CompilerParams(use_tc_tiling_on_sc=False),
)


def kernel(user_emb, item_emb, adj_values, adj_indices):
    row = adj_indices[0].astype(jnp.int32)
    col = adj_indices[1].astype(jnp.int32)
    val = adj_values.astype(jnp.float32)

    pad = EP - N_EDGES
    colp = jnp.concatenate([col, jnp.zeros((pad,), jnp.int32)])
    lidxp = jnp.concatenate([row, jnp.zeros((pad,), jnp.int32)])
    valp = jnp.concatenate([val, jnp.zeros((pad,), jnp.float32)])
    offs = (jnp.arange(NQ, dtype=jnp.int32) * N_NODES)[:, None]
    colh = (colp[None, :] + offs).reshape(NQ, EP // CH, CH)
    lidxh = lidxp.reshape(EP // CH, CH)

    ego = jnp.concatenate([user_emb, item_emb], axis=0)
    table = jnp.concatenate(
        [ego[:, 0:16], ego[:, 16:32], ego[:, 32:48], ego[:, 48:64]], axis=0)

    zrows = jnp.zeros((PT, QC), jnp.float32)
    flag0 = jnp.zeros((16,), jnp.int32)
    flag1 = jnp.ones((16,), jnp.int32)

    t1 = _spmm(table, colh, lidxh, valp, flag0, zrows, table, table)
    t2 = _spmm(t1, colh, lidxh, valp, flag0, zrows, table, table)
    m = _spmm(t2, colh, lidxh, valp, flag1, zrows, t1, t2)

    user = jnp.concatenate(
        [m[i * N_NODES:i * N_NODES + USER_NUM] for i in range(NQ)], axis=1)
    item = jnp.concatenate(
        [m[i * N_NODES + USER_NUM:(i + 1) * N_NODES] for i in range(NQ)],
        axis=1)
    return (user, item)


# speedup vs baseline: 1.7758x; 1.0553x over previous
import jax
import jax.numpy as jnp
from jax import lax
from jax.experimental import pallas as pl
from jax.experimental.pallas import tpu as pltpu
from jax.experimental.pallas import tpu_sc as plsc

USER_NUM = 25000
ITEM_NUM = 25000
N_NODES = USER_NUM + ITEM_NUM
N_EDGES = 800000
EMB = 64
QC = 16
NQ = 4
N_LAYERS = 3

NC = 2
NS = 16
CH = 128
JJ = 8
GROUP = JJ * CH
G = 49
EPT = G * GROUP
EP = NS * EPT
PT = N_NODES // NS


def _spmm_body(table, colh, lidxh, valh, zrows, out,
               colv, lidxv, valv, gbuf, acc, gsem, ssem):
    c = lax.axis_index("c")
    s = lax.axis_index("s")

    row_base = s * (EPT // CH)
    flat_base = s * EPT

    def drain_scatter(j):
        pltpu.make_async_copy(
            gbuf.at[pl.ds(j * CH, CH)],
            acc.at[pl.ds(j * CH, CH)], ssem.at[j]).wait()

    for p in range(2):
        q = c * 2 + p

        pltpu.sync_copy(zrows, acc.at[pl.ds(s * PT, PT)])
        plsc.subcore_barrier()

        def group_loop(g, _):
            @pl.when(g > 0)
            def _():
                for j in range(JJ):
                    drain_scatter(j)

            roff = row_base + g * JJ
            foff = flat_base + g * GROUP
            pltpu.sync_copy(colh.at[q, pl.ds(roff, JJ)], colv)
            pltpu.sync_copy(lidxh.at[pl.ds(roff, JJ)], lidxv)
            pltpu.sync_copy(valh.at[pl.ds(foff, GROUP)], valv)

            gds = [pltpu.async_copy(table.at[colv.at[j]],
                                    gbuf.at[pl.ds(j * CH, CH)], gsem.at[j])
                   for j in range(JJ)]

            for j in range(JJ):
                gds[j].wait()

                def e_loop(e16, _):
                    vv = valv[pl.ds(j * CH + e16 * 16, 16)]
                    for l in range(16):
                        r = j * CH + e16 * 16 + l
                        gbuf[r, :] = gbuf[r, :] * vv[l]
                    return 0

                lax.fori_loop(0, CH // 16, e_loop, 0)
                pltpu.async_copy(gbuf.at[pl.ds(j * CH, CH)],
                                 acc.at[lidxv.at[j]], ssem.at[j], add=True)
            return 0

        lax.fori_loop(0, G, group_loop, 0)
        for j in range(JJ):
            drain_scatter(j)
        plsc.subcore_barrier()

        pltpu.sync_copy(acc.at[pl.ds(s * PT, PT)],
                        out.at[pl.ds(q * N_NODES + s * PT, PT)])


_spmm = pl.kernel(
    _spmm_body,
    out_type=jax.ShapeDtypeStruct((NQ * N_NODES, QC), jnp.float32),
    mesh=plsc.VectorSubcoreMesh(core_axis_name="c", subcore_axis_name="s"),
    scratch_types=[
        pltpu.VMEM((JJ, CH), jnp.int32),
        pltpu.VMEM((JJ, CH), jnp.int32),
        pltpu.VMEM((GROUP,), jnp.float32),
        pltpu.VMEM((GROUP, QC), jnp.float32),
        pltpu.VMEM_SHARED((N_NODES, QC), jnp.float32),
        pltpu.SemaphoreType.DMA((JJ,)),
        pltpu.SemaphoreType.DMA((JJ,)),
    ],
    compiler_params=pltpu.CompilerParams(use_tc_tiling_on_sc=False),
)


def _mean3_body(a, b, c, o):
    o[...] = (a[...] + b[...] + c[...]) * (1.0 / 3.0)


def _mean3(a, b, c):
    rows = NQ * N_NODES * QC // 128
    a = a.reshape(rows, 128)
    b = b.reshape(rows, 128)
    c = c.reshape(rows, 128)
    blk = 1000
    spec = pl.BlockSpec((blk, 128), lambda i: (i, 0))
    out = pl.pallas_call(
        _mean3_body,
        grid=(rows // blk,),
        in_specs=[spec, spec, spec],
        out_specs=spec,
        out_shape=jax.ShapeDtypeStruct((rows, 128), jnp.float32),
    )(a, b, c)
    return out.reshape(NQ * N_NODES, QC)


def kernel(user_emb, item_emb, adj_values, adj_indices):
    row = adj_indices[0].astype(jnp.int32)
    col = adj_indices[1].astype(jnp.int32)
    val = adj_values.astype(jnp.float32)

    pad = EP - N_EDGES
    colp = jnp.concatenate([col, jnp.zeros((pad,), jnp.int32)])
    lidxp = jnp.concatenate([row, jnp.zeros((pad,), jnp.int32)])
    valp = jnp.concatenate([val, jnp.zeros((pad,), jnp.float32)])
    offs = (jnp.arange(NQ, dtype=jnp.int32) * N_NODES)[:, None]
    colh = (colp[None, :] + offs).reshape(NQ, EP // CH, CH)
    lidxh = lidxp.reshape(EP // CH, CH)

    ego = jnp.concatenate([user_emb, item_emb], axis=0)
    table = jnp.concatenate(
        [ego[:, 0:16], ego[:, 16:32], ego[:, 32:48], ego[:, 48:64]], axis=0)

    zrows = jnp.zeros((PT, QC), jnp.float32)

    layers = []
    for _ in range(N_LAYERS):
        table = _spmm(table, colh, lidxh, valp, zrows)
        layers.append(table)

    m = _mean3(*layers)

    user = jnp.concatenate(
        [m[i * N_NODES:i * N_NODES + USER_NUM] for i in range(NQ)], axis=1)
    item = jnp.concatenate(
        [m[i * N_NODES + USER_NUM:(i + 1) * N_NODES] for i in range(NQ)],
        axis=1)
    return (user, item)
